# Initial kernel scaffold; baseline (speedup 1.0000x reference)
import jax
import jax.numpy as jnp
from jax.experimental import pallas as pl


def kernel(nodes, features, edges, W1, W2, W3, W4, W5, W6, W7, Wl, bl):
    raise NotImplementedError("write your pallas kernel here")


# speedup vs baseline: 1.7382x; 1.7382x over previous
import jax
import jax.numpy as jnp
from jax.experimental import pallas as pl

INPUT_DIM = 256
G1 = 64
V1ROWS = G1 * G1 * G1
G2 = 16
V2ROWS = G2 * G2 * G2


def _fused_conv(x, agg, W, block_rows):
    R, Din = x.shape
    Dout = W.shape[1]

    def body(x_ref, a_ref, w_ref, o_ref):
        xa = x_ref[...] + a_ref[...]
        o_ref[...] = jnp.maximum(
            jnp.dot(xa, w_ref[...], preferred_element_type=jnp.float32), 0.0)

    return pl.pallas_call(
        body,
        grid=(R // block_rows,),
        in_specs=[
            pl.BlockSpec((block_rows, Din), lambda i: (i, 0)),
            pl.BlockSpec((block_rows, Din), lambda i: (i, 0)),
            pl.BlockSpec((Din, Dout), lambda i: (0, 0)),
        ],
        out_specs=pl.BlockSpec((block_rows, Dout), lambda i: (i, 0)),
        out_shape=jax.ShapeDtypeStruct((R, Dout), jnp.float32),
    )(x, agg, W)


def _conv_dense_adj(A, z, W, block_rows=512):
    R, Din = z.shape
    Dout = W.shape[1]

    def body(a_ref, zb_ref, zf_ref, w_ref, o_ref):
        agg = jnp.dot(a_ref[...], zf_ref[...],
                      preferred_element_type=jnp.float32)
        o_ref[...] = jnp.maximum(
            jnp.dot(zb_ref[...] + agg, w_ref[...],
                    preferred_element_type=jnp.float32), 0.0)

    return pl.pallas_call(
        body,
        grid=(R // block_rows,),
        in_specs=[
            pl.BlockSpec((block_rows, R), lambda i: (i, 0)),
            pl.BlockSpec((block_rows, Din), lambda i: (i, 0)),
            pl.BlockSpec((R, Din), lambda i: (0, 0)),
            pl.BlockSpec((Din, Dout), lambda i: (0, 0)),
        ],
        out_specs=pl.BlockSpec((block_rows, Dout), lambda i: (i, 0)),
        out_shape=jax.ShapeDtypeStruct((R, Dout), jnp.float32),
    )(A, z, z, W)


def _maxpool_4x4x4(x, g, C, block):
    h = g // 4
    xg = x.reshape(h, 4, h, 4, h, 4, C)
    xg = xg.transpose(0, 2, 4, 1, 3, 5, 6).reshape(h * h * h, 64 * C)

    def body(x_ref, o_ref):
        blk = x_ref[...].reshape(block, 64, C)
        o_ref[...] = jnp.max(blk, axis=1)

    return pl.pallas_call(
        body,
        grid=(h * h * h // block,),
        in_specs=[pl.BlockSpec((block, 64 * C), lambda i: (i, 0))],
        out_specs=pl.BlockSpec((block, C), lambda i: (i, 0)),
        out_shape=jax.ShapeDtypeStruct((h * h * h, C), jnp.float32),
    )(xg)


def _final_linear(flat, Wl, bl):
    def body(f_ref, w_ref, b_ref, o_ref):
        o_ref[...] = jnp.dot(
            f_ref[...], w_ref[...],
            preferred_element_type=jnp.float32) + b_ref[...]

    out = pl.pallas_call(
        body,
        out_shape=jax.ShapeDtypeStruct((8, Wl.shape[1]), jnp.float32),
    )(jnp.broadcast_to(flat[None, :], (8, flat.shape[0])), Wl,
      jnp.broadcast_to(bl[None, :], (8, bl.shape[0])))
    return out[0]


def kernel(nodes, features, edges, W1, W2, W3, W4, W5, W6, W7, Wl, bl):
    src = edges[0]
    dst = edges[1]
    N = features.shape[0]

    agg1 = jax.ops.segment_sum(features[src], dst, num_segments=N)
    x1 = _fused_conv(features, agg1, W1, 2000)
    agg2 = jax.ops.segment_sum(x1[src], dst, num_segments=N)
    x2 = _fused_conv(x1, agg2, W2, 2000)

    c = nodes // 4
    cl1 = (c[:, 0] * G1 + c[:, 1]) * G1 + c[:, 2]
    v1 = jax.ops.segment_max(x2, cl1, num_segments=V1ROWS)
    v1 = jnp.where(jnp.isfinite(v1), v1, 0.0)
    e1s = cl1[src]
    e1d = cl1[dst]

    agg3 = jax.ops.segment_sum(v1[e1s], e1d, num_segments=V1ROWS)
    y1 = _fused_conv(v1, agg3, W3, 2048)
    agg4 = jax.ops.segment_sum(y1[e1s], e1d, num_segments=V1ROWS)
    y2 = _fused_conv(y1, agg4, W4, 2048)

    v2 = _maxpool_4x4x4(y2, G1, 32, 512)
    c0 = e1s // (G1 * G1)
    c1 = (e1s // G1) % G1
    c2 = e1s % G1
    e2s = ((c0 // 4) * G2 + (c1 // 4)) * G2 + (c2 // 4)
    c0 = e1d // (G1 * G1)
    c1 = (e1d // G1) % G1
    c2 = e1d % G1
    e2d = ((c0 // 4) * G2 + (c1 // 4)) * G2 + (c2 // 4)

    A2 = jnp.zeros((V2ROWS, V2ROWS), jnp.float32).at[e2d, e2s].add(1.0)
    z1 = _conv_dense_adj(A2, v2, W5)
    z2 = _conv_dense_adj(A2, z1, W6)
    z3 = _conv_dense_adj(A2, z2, W7)

    flat = _maxpool_4x4x4(z3, G2, 64, 64).reshape(-1)
    return _final_linear(flat, Wl, bl)


# speedup vs baseline: 12.4750x; 7.1771x over previous
import functools

import jax
import jax.numpy as jnp
from jax import lax
from jax.experimental import pallas as pl
from jax.experimental.pallas import tpu as pltpu
from jax.experimental.pallas import tpu_sc as plsc

INPUT_DIM = 256
G1 = 64
V1ROWS = G1 * G1 * G1
G2 = 16
V2ROWS = G2 * G2 * G2

E_PAD_G = 6400
KPAD = 51200
TRASH = 2048
NC, NS = 2, 16


def _sc_mesh():
    return plsc.VectorSubcoreMesh(
        core_axis_name="c", subcore_axis_name="s",
        num_cores=NC, num_subcores=NS)


def _sc_segsum(x, src2d, dst2d, ch, gpb=8):
    d = x.shape[1]
    groups = src2d.shape[0]
    gpt = groups // (NC * NS)
    nbuf = gpt // gpb
    zr = (ch + TRASH) // NS
    outr = ch // NS
    zeros = jnp.zeros((zr, d), jnp.float32)

    @functools.partial(
        pl.kernel,
        out_type=jax.ShapeDtypeStruct((2 * ch, d), jnp.float32),
        mesh=_sc_mesh(),
        compiler_params=pltpu.CompilerParams(use_tc_tiling_on_sc=False),
        scratch_types=[
            pltpu.VMEM((gpb, 128), jnp.int32),
            pltpu.VMEM((gpb, 128), jnp.int32),
            pltpu.VMEM((gpb, 128), jnp.int32),
            pltpu.VMEM((gpb, 128, d), jnp.float32),
            pltpu.VMEM_SHARED((ch + TRASH, d), jnp.float32),
            pltpu.SemaphoreType.DMA,
        ],
    )
    def k(x_hbm, src_hbm, dst_hbm, z_hbm, out_hbm,
          sbuf, dbuf, ldb, rows, acc, sem):
        cc = lax.axis_index("c")
        ss = lax.axis_index("s")
        wid = cc * NS + ss
        pltpu.sync_copy(z_hbm, acc.at[pl.ds(ss * zr, zr)])
        plsc.subcore_barrier()

        def buf_body(b, carry):
            g0 = wid * gpt + b * gpb
            pltpu.sync_copy(src_hbm.at[pl.ds(g0, gpb)], sbuf)
            pltpu.sync_copy(dst_hbm.at[pl.ds(g0, gpb)], dbuf)
            ji = lax.iota(jnp.int32, 16)
            for r in range(gpb):
                def cbody(ci, c2):
                    dd = dbuf[r, pl.ds(ci * 16, 16)]
                    ok = (dd >= 0) & (dd < ch)
                    spread = (dd + ci * 16 + ji) & (TRASH - 1)
                    ldb[r, pl.ds(ci * 16, 16)] = jnp.where(
                        ok, dd, ch + spread)
                    return c2
                lax.fori_loop(0, 8, cbody, 0)
            cps = [pltpu.async_copy(x_hbm.at[sbuf.at[r]], rows.at[r], sem)
                   for r in range(gpb)]
            for r in range(gpb):
                cps[r].wait()
                pltpu.sync_copy(rows.at[r], acc.at[ldb.at[r]], add=True)
            return carry
        lax.fori_loop(0, nbuf, buf_body, 0)
        plsc.subcore_barrier()
        pltpu.sync_copy(acc.at[pl.ds(ss * outr, outr)],
                        out_hbm.at[pl.ds(cc * ch + ss * outr, outr)])

    return k(x, src2d, dst2d, zeros)


def _sc_edge_gather(t1, t2, src2d, dst2d, gpb=8):
    groups = src2d.shape[0]
    gpt = groups // (NC * NS)
    nbuf = gpt // gpb
    ot = jax.ShapeDtypeStruct((groups, 128), jnp.int32)

    @functools.partial(
        pl.kernel,
        out_type=(ot, ot, ot, ot),
        mesh=_sc_mesh(),
        compiler_params=pltpu.CompilerParams(use_tc_tiling_on_sc=False),
        scratch_types=[
            pltpu.VMEM((gpb, 128), jnp.int32),
            pltpu.VMEM((gpb, 128), jnp.int32),
            pltpu.VMEM((gpb, 128), jnp.int32),
            pltpu.VMEM((gpb, 128), jnp.int32),
            pltpu.VMEM((gpb, 128), jnp.int32),
            pltpu.VMEM((gpb, 128), jnp.int32),
            pltpu.SemaphoreType.DMA,
        ],
    )
    def k(t1_hbm, t2_hbm, src_hbm, dst_hbm,
          o_ss, o_sd, o_cs, o_cd,
          sbuf, dbuf, ga, gb, gc, gd, sem):
        cc = lax.axis_index("c")
        ss = lax.axis_index("s")
        wid = cc * NS + ss

        def buf_body(b, carry):
            g0 = wid * gpt + b * gpb
            pltpu.sync_copy(src_hbm.at[pl.ds(g0, gpb)], sbuf)
            pltpu.sync_copy(dst_hbm.at[pl.ds(g0, gpb)], dbuf)
            for r in range(gpb):
                def cbody(ci, c2):
                    dd = dbuf[r, pl.ds(ci * 16, 16)]
                    dbuf[r, pl.ds(ci * 16, 16)] = jnp.maximum(dd, 0)
                    return c2
                lax.fori_loop(0, 8, cbody, 0)
            cps = []
            for r in range(gpb):
                cps.append(pltpu.async_copy(
                    t1_hbm.at[sbuf.at[r]], ga.at[r], sem))
                cps.append(pltpu.async_copy(
                    t1_hbm.at[dbuf.at[r]], gb.at[r], sem))
            cps2 = []
            for r in range(gpb):
                cps[2 * r].wait()
                cps2.append(pltpu.async_copy(
                    t2_hbm.at[ga.at[r]], gc.at[r], sem))
                cps[2 * r + 1].wait()
                cps2.append(pltpu.async_copy(
                    t2_hbm.at[gb.at[r]], gd.at[r], sem))
            for c in cps2:
                c.wait()
            pltpu.sync_copy(ga, o_ss.at[pl.ds(g0, gpb)])
            pltpu.sync_copy(gb, o_sd.at[pl.ds(g0, gpb)])
            pltpu.sync_copy(gc, o_cs.at[pl.ds(g0, gpb)])
            pltpu.sync_copy(gd, o_cd.at[pl.ds(g0, gpb)])
            return carry
        lax.fori_loop(0, nbuf, buf_body, 0)

    return k(t1, t2, src2d, dst2d)


def _sc_gather1(table, idx2d, gpb=8):
    groups = idx2d.shape[0]
    gpt = groups // (NC * NS)
    nbuf = gpt // gpb

    @functools.partial(
        pl.kernel,
        out_type=jax.ShapeDtypeStruct((groups, 128), jnp.int32),
        mesh=_sc_mesh(),
        compiler_params=pltpu.CompilerParams(use_tc_tiling_on_sc=False),
        scratch_types=[
            pltpu.VMEM((gpb, 128), jnp.int32),
            pltpu.VMEM((gpb, 128), jnp.int32),
            pltpu.SemaphoreType.DMA,
        ],
    )
    def k(t_hbm, idx_hbm, out_hbm, ibuf, gbuf, sem):
        cc = lax.axis_index("c")
        ss = lax.axis_index("s")
        wid = cc * NS + ss

        def buf_body(b, carry):
            g0 = wid * gpt + b * gpb
            pltpu.sync_copy(idx_hbm.at[pl.ds(g0, gpb)], ibuf)
            cps = [pltpu.async_copy(t_hbm.at[ibuf.at[r]], gbuf.at[r], sem)
                   for r in range(gpb)]
            for c in cps:
                c.wait()
            pltpu.sync_copy(gbuf, out_hbm.at[pl.ds(g0, gpb)])
            return carry
        lax.fori_loop(0, nbuf, buf_body, 0)

    return k(table, idx2d)


def _fused_conv(x, a0, a1, W, block_rows):
    R, Din = x.shape
    Dout = W.shape[1]

    def body(x_ref, a_ref, b_ref, w_ref, o_ref):
        xa = x_ref[...] + a_ref[...] + b_ref[...]
        o_ref[...] = jnp.maximum(
            jnp.dot(xa, w_ref[...], preferred_element_type=jnp.float32), 0.0)

    return pl.pallas_call(
        body,
        grid=(R // block_rows,),
        in_specs=[
            pl.BlockSpec((block_rows, Din), lambda i: (i, 0)),
            pl.BlockSpec((block_rows, Din), lambda i: (i, 0)),
            pl.BlockSpec((block_rows, Din), lambda i: (i, 0)),
            pl.BlockSpec((Din, Dout), lambda i: (0, 0)),
        ],
        out_specs=pl.BlockSpec((block_rows, Dout), lambda i: (i, 0)),
        out_shape=jax.ShapeDtypeStruct((R, Dout), jnp.float32),
    )(x, a0, a1, W)


def _conv_dense_adj(A, z, W, block_rows=512):
    R, Din = z.shape
    Dout = W.shape[1]

    def body(a_ref, zb_ref, zf_ref, w_ref, o_ref):
        agg = jnp.dot(a_ref[...], zf_ref[...],
                      preferred_element_type=jnp.float32)
        o_ref[...] = jnp.maximum(
            jnp.dot(zb_ref[...] + agg, w_ref[...],
                    preferred_element_type=jnp.float32), 0.0)

    return pl.pallas_call(
        body,
        grid=(R // block_rows,),
        in_specs=[
            pl.BlockSpec((block_rows, R), lambda i: (i, 0)),
            pl.BlockSpec((block_rows, Din), lambda i: (i, 0)),
            pl.BlockSpec((R, Din), lambda i: (0, 0)),
            pl.BlockSpec((Din, Dout), lambda i: (0, 0)),
        ],
        out_specs=pl.BlockSpec((block_rows, Dout), lambda i: (i, 0)),
        out_shape=jax.ShapeDtypeStruct((R, Dout), jnp.float32),
    )(A, z, z, W)


def _maxpool_4x4x4(x, g, C, block):
    h = g // 4
    xg = x.reshape(h, 4, h, 4, h, 4, C)
    xg = xg.transpose(0, 2, 4, 1, 3, 5, 6).reshape(h * h * h, 64 * C)

    def body(x_ref, o_ref):
        blk = x_ref[...].reshape(block, 64, C)
        o_ref[...] = jnp.max(blk, axis=1)

    return pl.pallas_call(
        body,
        grid=(h * h * h // block,),
        in_specs=[pl.BlockSpec((block, 64 * C), lambda i: (i, 0))],
        out_specs=pl.BlockSpec((block, C), lambda i: (i, 0)),
        out_shape=jax.ShapeDtypeStruct((h * h * h, C), jnp.float32),
    )(xg)


def _final_linear(flat, Wl, bl):
    def body(f_ref, w_ref, b_ref, o_ref):
        o_ref[...] = jnp.dot(
            f_ref[...], w_ref[...],
            preferred_element_type=jnp.float32) + b_ref[...]

    out = pl.pallas_call(
        body,
        out_shape=jax.ShapeDtypeStruct((8, Wl.shape[1]), jnp.float32),
    )(jnp.broadcast_to(flat[None, :], (8, flat.shape[0])), Wl,
      jnp.broadcast_to(bl[None, :], (8, bl.shape[0])))
    return out[0]


def _spatial_to_pool2(e):
    c0 = e // (G1 * G1)
    c1 = (e // G1) % G1
    c2 = e % G1
    return ((c0 // 4) * G2 + (c1 // 4)) * G2 + (c2 // 4)


def kernel(nodes, features, edges, W1, W2, W3, W4, W5, W6, W7, Wl, bl):
    N = features.shape[0]
    e_pad = E_PAD_G * 128
    src2d = jnp.concatenate(
        [edges[0], jnp.zeros((e_pad - edges.shape[1],), jnp.int32)]
    ).reshape(E_PAD_G, 128)
    dst2d = jnp.concatenate(
        [edges[1], jnp.full((e_pad - edges.shape[1],), -1, jnp.int32)]
    ).reshape(E_PAD_G, 128)

    nch = 50176
    p1 = _sc_segsum(jnp.tile(features, (1, 8)), src2d, dst2d, nch)
    x1 = _fused_conv(features, p1[:N, :1], p1[nch:nch + N, :1], W1, 2000)
    p2 = _sc_segsum(x1, src2d, dst2d, nch)
    x2 = _fused_conv(x1, p2[:N], p2[nch:nch + N], W2, 2000)

    c = nodes // 4
    cl1 = ((c[:, 0] * G1 + c[:, 1]) * G1 + c[:, 2]).astype(jnp.int32)
    srt = jnp.sort(cl1)
    newflag = jnp.concatenate(
        [jnp.ones((1,), jnp.int32), (srt[1:] != srt[:-1]).astype(jnp.int32)])
    rank = jnp.cumsum(newflag) - 1
    crank = jnp.zeros((V1ROWS,), jnp.int32).at[srt].set(rank)
    uids = jnp.zeros((KPAD,), jnp.int32).at[rank].set(srt)

    e_ss, e_sd, e_cs, e_cd = _sc_edge_gather(cl1, crank, src2d, dst2d)
    e_cd = jnp.where(dst2d < 0, -1, e_cd)

    n_pad_g = 512
    cl1_2d = jnp.concatenate(
        [cl1, jnp.zeros((n_pad_g * 128 - N,), jnp.int32)]).reshape(n_pad_g, 128)
    inv = _sc_gather1(crank, cl1_2d).reshape(-1)[:N]
    v1 = jax.ops.segment_max(x2, inv, num_segments=KPAD)
    v1 = jnp.where(jnp.isfinite(v1), v1, 0.0)

    p3 = _sc_segsum(v1, e_cs, e_cd, KPAD)
    y1 = _fused_conv(v1, p3[:KPAD], p3[KPAD:], W3, 2048)
    p4a = _sc_segsum(y1[:, :16], e_cs, e_cd, KPAD)
    p4b = _sc_segsum(y1[:, 16:], e_cs, e_cd, KPAD)
    p4 = jnp.concatenate([p4a, p4b], axis=1)
    y2 = _fused_conv(y1, p4[:KPAD], p4[KPAD:], W4, 2048)

    cmap = _spatial_to_pool2(uids)
    v2 = jax.ops.segment_max(y2, cmap, num_segments=V2ROWS)
    v2 = jnp.where(jnp.isfinite(v2), v2, 0.0)
    n_e = edges.shape[1]
    e2s = _spatial_to_pool2(e_ss).reshape(-1)[:n_e]
    e2d = _spatial_to_pool2(e_sd).reshape(-1)[:n_e]

    A2 = jnp.zeros((V2ROWS, V2ROWS), jnp.float32).at[e2d, e2s].add(1.0)
    z1 = _conv_dense_adj(A2, v2, W5)
    z2 = _conv_dense_adj(A2, z1, W6)
    z3 = _conv_dense_adj(A2, z2, W7)

    flat = _maxpool_4x4x4(z3, G2, 64, 64).reshape(-1)
    return _final_linear(flat, Wl, bl)
